# Initial kernel scaffold; baseline (speedup 1.0000x reference)
import functools

import jax
import jax.numpy as jnp
from jax import lax
from jax.experimental import pallas as pl
from jax.experimental.pallas import tpu as pltpu
from jax.experimental.pallas import tpu_sc as plsc

N = 320000
C = 128
S = 2048

NUM_CORES = 2
NUM_SUBCORES = 16
NW = NUM_CORES * NUM_SUBCORES
ROWS_PER_W = N // NW
CHUNK = 400
NCHUNK = ROWS_PER_W // CHUNK
SUB = 80
NSUB = CHUNK // SUB
GROUPS = N // SUB

_mesh = plsc.VectorSubcoreMesh(core_axis_name="c", subcore_axis_name="s")


@functools.partial(
    pl.kernel,
    out_type=(
        jax.ShapeDtypeStruct((NUM_CORES, S, C), jnp.float32),
        jax.ShapeDtypeStruct((NW, S), jnp.float32),
    ),
    mesh=_mesh,
    scratch_types=[
        pltpu.VMEM((CHUNK, C), jnp.float32),
        pltpu.VMEM((NSUB, SUB), jnp.int32),
        pltpu.VMEM((S,), jnp.float32),
        pltpu.VMEM_SHARED((S, C), jnp.float32),
    ],
)
def _phase_a(x_hbm, b2d_hbm, zeros_hbm, psums_hbm, pcnt_hbm,
             xbuf, idxbuf, cnt, sums_sh):
    cid = lax.axis_index("c")
    sid = lax.axis_index("s")
    wid = cid * NUM_SUBCORES + sid

    rows_per_tile = S // NUM_SUBCORES
    pltpu.sync_copy(zeros_hbm.at[pl.ds(sid * rows_per_tile, rows_per_tile)],
                    sums_sh.at[pl.ds(sid * rows_per_tile, rows_per_tile)])
    zeros16 = jnp.zeros((16,), jnp.float32)

    def zero_body(i, carry):
        cnt[pl.ds(i * 16, 16)] = zeros16
        return carry

    lax.fori_loop(0, S // 16, zero_body, 0)
    plsc.subcore_barrier()

    ones16 = jnp.ones((16,), jnp.float32)

    def chunk_body(k, carry):
        row0 = wid * ROWS_PER_W + k * CHUNK
        g0 = wid * (ROWS_PER_W // SUB) + k * NSUB
        pltpu.sync_copy(x_hbm.at[pl.ds(row0, CHUNK)], xbuf)
        pltpu.sync_copy(b2d_hbm.at[pl.ds(g0, NSUB)], idxbuf)
        for j in range(NSUB):
            pltpu.sync_copy(xbuf.at[pl.ds(j * SUB, SUB)],
                            sums_sh.at[idxbuf.at[j]], add=True)
            for t in range(SUB // 16):
                idx16 = idxbuf[j, pl.ds(t * 16, 16)]
                plsc.addupdate_scatter(cnt, [idx16], ones16)
        return carry

    lax.fori_loop(0, NCHUNK, chunk_body, 0)

    plsc.subcore_barrier()
    pltpu.sync_copy(sums_sh.at[pl.ds(sid * rows_per_tile, rows_per_tile)],
                    psums_hbm.at[cid, pl.ds(sid * rows_per_tile, rows_per_tile)])
    pltpu.sync_copy(cnt, pcnt_hbm.at[wid])


@functools.partial(
    pl.kernel,
    out_type=jax.ShapeDtypeStruct((NUM_CORES, S, C), jnp.float32),
    mesh=_mesh,
    scratch_types=[
        pltpu.VMEM((CHUNK, C), jnp.float32),
        pltpu.VMEM((CHUNK, C), jnp.float32),
        pltpu.VMEM((NSUB, SUB), jnp.int32),
        pltpu.VMEM_SHARED((S, C), jnp.float32),
        pltpu.SemaphoreType.DMA,
    ],
)
def _phase_b(x_hbm, b2d_hbm, c_hbm, zeros_hbm, hpart_hbm,
             xbuf, crows, idxbuf, h_sh, sem):
    cid = lax.axis_index("c")
    sid = lax.axis_index("s")
    wid = cid * NUM_SUBCORES + sid

    rows_per_tile = S // NUM_SUBCORES
    pltpu.sync_copy(zeros_hbm.at[pl.ds(sid * rows_per_tile, rows_per_tile)],
                    h_sh.at[pl.ds(sid * rows_per_tile, rows_per_tile)])
    plsc.subcore_barrier()

    def chunk_body(k, carry):
        row0 = wid * ROWS_PER_W + k * CHUNK
        g0 = wid * (ROWS_PER_W // SUB) + k * NSUB
        pltpu.sync_copy(x_hbm.at[pl.ds(row0, CHUNK)], xbuf)
        pltpu.sync_copy(b2d_hbm.at[pl.ds(g0, NSUB)], idxbuf)
        for j in range(NSUB):
            pltpu.async_copy(c_hbm.at[idxbuf.at[j]],
                             crows.at[pl.ds(j * SUB, SUB)], sem).wait()

        def row_body(r, rcarry):
            xs = [xbuf[r, pl.ds(16 * q, 16)] for q in range(C // 16)]
            cs = [crows[r, pl.ds(16 * q, 16)] for q in range(C // 16)]
            acc = xs[0] * cs[0]
            for q in range(1, C // 16):
                acc = acc + xs[q] * cs[q]
            z = jnp.sum(acc)
            zv = jnp.full((16,), z, jnp.float32)
            gate = 1.0 / (1.0 + jnp.exp(-zv))
            for q in range(C // 16):
                xbuf[r, pl.ds(16 * q, 16)] = xs[q] * gate
            return rcarry

        lax.fori_loop(0, CHUNK, row_body, 0)

        for j in range(NSUB):
            pltpu.sync_copy(xbuf.at[pl.ds(j * SUB, SUB)],
                            h_sh.at[idxbuf.at[j]], add=True)
        return carry

    lax.fori_loop(0, NCHUNK, chunk_body, 0)

    plsc.subcore_barrier()
    pltpu.sync_copy(h_sh.at[pl.ds(sid * rows_per_tile, rows_per_tile)],
                    hpart_hbm.at[cid, pl.ds(sid * rows_per_tile, rows_per_tile)])


def _mid_body(ps_ref, pc_ref, w_ref, c_ref):
    sums = ps_ref[0] + ps_ref[1]
    counts = jnp.sum(pc_ref[...], axis=0)
    mean = sums / jnp.maximum(counts, 1.0)[:, None]
    c_ref[...] = jnp.tanh(
        jnp.dot(mean, w_ref[...], preferred_element_type=jnp.float32))


def _add_body(hp_ref, out_ref):
    out_ref[...] = hp_ref[0] + hp_ref[1]


def kernel(x, batch, weight_c):
    batch = batch.astype(jnp.int32)
    b2d = batch.reshape(GROUPS, SUB)
    zeros = jnp.zeros((S, C), jnp.float32)

    psums, pcnt = _phase_a(x, b2d, zeros)

    c = pl.pallas_call(
        _mid_body,
        out_shape=jax.ShapeDtypeStruct((S, C), jnp.float32),
    )(psums, pcnt, weight_c)

    hpart = _phase_b(x, b2d, c, zeros)

    h = pl.pallas_call(
        _add_body,
        out_shape=jax.ShapeDtypeStruct((S, C), jnp.float32),
    )(hpart)
    return h


# speedup vs baseline: 2.1165x; 2.1165x over previous
import functools

import jax
import jax.numpy as jnp
from jax import lax
from jax.experimental import pallas as pl
from jax.experimental.pallas import tpu as pltpu
from jax.experimental.pallas import tpu_sc as plsc

N = 320000
C = 128
S = 2048

NUM_CORES = 2
NUM_SUBCORES = 16
NW = NUM_CORES * NUM_SUBCORES
ROWS_PER_W = N // NW
CHUNK = 400
NCHUNK = ROWS_PER_W // CHUNK
SUB = 80
NSUB = CHUNK // SUB

_mesh = plsc.VectorSubcoreMesh(core_axis_name="c", subcore_axis_name="s")
_sc_params = pltpu.CompilerParams(needs_layout_passes=False)


def _stage_idx2d(idx1d, idx2d):
    for j in range(NSUB):
        for t in range(SUB // 16):
            idx2d[j, pl.ds(t * 16, 16)] = idx1d[pl.ds(j * SUB + t * 16, 16)]


@functools.partial(
    pl.kernel,
    out_type=(
        jax.ShapeDtypeStruct((NUM_CORES, S, C), jnp.float32),
        jax.ShapeDtypeStruct((NW * S,), jnp.float32),
    ),
    mesh=_mesh,
    scratch_types=[
        pltpu.VMEM((CHUNK, C), jnp.float32),
        pltpu.VMEM((CHUNK,), jnp.int32),
        pltpu.VMEM((NSUB, SUB), jnp.int32),
        pltpu.VMEM((S,), jnp.float32),
        pltpu.VMEM_SHARED((S, C), jnp.float32),
    ],
    compiler_params=_sc_params,
)
def _phase_a(x_hbm, b_hbm, zeros_hbm, psums_hbm, pcnt_hbm,
             xbuf, idx1d, idx2d, cnt, sums_sh):
    cid = lax.axis_index("c")
    sid = lax.axis_index("s")
    wid = cid * NUM_SUBCORES + sid

    rows_per_tile = S // NUM_SUBCORES
    pltpu.sync_copy(zeros_hbm.at[pl.ds(sid * rows_per_tile, rows_per_tile)],
                    sums_sh.at[pl.ds(sid * rows_per_tile, rows_per_tile)])
    zeros16 = jnp.zeros((16,), jnp.float32)

    def zero_body(i, carry):
        cnt[pl.ds(i * 16, 16)] = zeros16
        return carry

    lax.fori_loop(0, S // 16, zero_body, 0)
    plsc.subcore_barrier()

    ones16 = jnp.ones((16,), jnp.float32)

    def chunk_body(k, carry):
        row0 = wid * ROWS_PER_W + k * CHUNK
        pltpu.sync_copy(x_hbm.at[pl.ds(row0, CHUNK)], xbuf)
        pltpu.sync_copy(b_hbm.at[pl.ds(row0, CHUNK)], idx1d)
        _stage_idx2d(idx1d, idx2d)
        for j in range(NSUB):
            pltpu.sync_copy(xbuf.at[pl.ds(j * SUB, SUB)],
                            sums_sh.at[idx2d.at[j]], add=True)
            for t in range(SUB // 16):
                idx16 = idx2d[j, pl.ds(t * 16, 16)]
                plsc.addupdate_scatter(cnt, [idx16], ones16)
        return carry

    lax.fori_loop(0, NCHUNK, chunk_body, 0)

    plsc.subcore_barrier()
    pltpu.sync_copy(sums_sh.at[pl.ds(sid * rows_per_tile, rows_per_tile)],
                    psums_hbm.at[cid, pl.ds(sid * rows_per_tile, rows_per_tile)])
    pltpu.sync_copy(cnt, pcnt_hbm.at[pl.ds(wid * S, S)])


@functools.partial(
    pl.kernel,
    out_type=jax.ShapeDtypeStruct((NUM_CORES, S, C), jnp.float32),
    mesh=_mesh,
    scratch_types=[
        pltpu.VMEM((CHUNK, C), jnp.float32),
        pltpu.VMEM((CHUNK, C), jnp.float32),
        pltpu.VMEM((CHUNK,), jnp.int32),
        pltpu.VMEM((NSUB, SUB), jnp.int32),
        pltpu.VMEM_SHARED((S, C), jnp.float32),
        pltpu.SemaphoreType.DMA,
    ],
    compiler_params=_sc_params,
)
def _phase_b(x_hbm, b_hbm, c_hbm, zeros_hbm, hpart_hbm,
             xbuf, crows, idx1d, idx2d, h_sh, sem):
    cid = lax.axis_index("c")
    sid = lax.axis_index("s")
    wid = cid * NUM_SUBCORES + sid

    rows_per_tile = S // NUM_SUBCORES
    pltpu.sync_copy(zeros_hbm.at[pl.ds(sid * rows_per_tile, rows_per_tile)],
                    h_sh.at[pl.ds(sid * rows_per_tile, rows_per_tile)])
    plsc.subcore_barrier()

    def chunk_body(k, carry):
        row0 = wid * ROWS_PER_W + k * CHUNK
        pltpu.sync_copy(x_hbm.at[pl.ds(row0, CHUNK)], xbuf)
        pltpu.sync_copy(b_hbm.at[pl.ds(row0, CHUNK)], idx1d)
        _stage_idx2d(idx1d, idx2d)
        for j in range(NSUB):
            pltpu.async_copy(c_hbm.at[idx2d.at[j]],
                             crows.at[pl.ds(j * SUB, SUB)], sem).wait()

        def row_body(r, rcarry):
            xs = [xbuf[r, pl.ds(16 * q, 16)] for q in range(C // 16)]
            cs = [crows[r, pl.ds(16 * q, 16)] for q in range(C // 16)]
            acc = xs[0] * cs[0]
            for q in range(1, C // 16):
                acc = acc + xs[q] * cs[q]
            z = jnp.sum(acc)
            zv = jnp.full((16,), z, jnp.float32)
            gate = 1.0 / (1.0 + jnp.exp(-zv))
            for q in range(C // 16):
                xbuf[r, pl.ds(16 * q, 16)] = xs[q] * gate
            return rcarry

        lax.fori_loop(0, CHUNK, row_body, 0)

        for j in range(NSUB):
            pltpu.sync_copy(xbuf.at[pl.ds(j * SUB, SUB)],
                            h_sh.at[idx2d.at[j]], add=True)
        return carry

    lax.fori_loop(0, NCHUNK, chunk_body, 0)

    plsc.subcore_barrier()
    pltpu.sync_copy(h_sh.at[pl.ds(sid * rows_per_tile, rows_per_tile)],
                    hpart_hbm.at[cid, pl.ds(sid * rows_per_tile, rows_per_tile)])


def _mid_body(ps_ref, pc_ref, w_ref, c_ref):
    sums = ps_ref[0] + ps_ref[1]
    counts = jnp.sum(pc_ref[...].reshape(NW, S), axis=0)
    mean = sums / jnp.maximum(counts, 1.0)[:, None]
    c_ref[...] = jnp.tanh(
        jnp.dot(mean, w_ref[...], preferred_element_type=jnp.float32))


def _add_body(hp_ref, out_ref):
    out_ref[...] = hp_ref[0] + hp_ref[1]


def kernel(x, batch, weight_c):
    batch = batch.astype(jnp.int32)
    zeros = jnp.zeros((S, C), jnp.float32)

    psums, pcnt = _phase_a(x, batch, zeros)

    c = pl.pallas_call(
        _mid_body,
        out_shape=jax.ShapeDtypeStruct((S, C), jnp.float32),
    )(psums, pcnt, weight_c)

    hpart = _phase_b(x, batch, c, zeros)

    h = pl.pallas_call(
        _add_body,
        out_shape=jax.ShapeDtypeStruct((S, C), jnp.float32),
    )(hpart)
    return h


# speedup vs baseline: 2.5244x; 1.1927x over previous
import functools

import jax
import jax.numpy as jnp
from jax import lax
from jax.experimental import pallas as pl
from jax.experimental.pallas import tpu as pltpu
from jax.experimental.pallas import tpu_sc as plsc

N = 320000
C = 128
S = 2048

NUM_CORES = 2
NUM_SUBCORES = 16
NW = NUM_CORES * NUM_SUBCORES
ROWS_PER_W = N // NW
CHUNK = 400
NCHUNK = ROWS_PER_W // CHUNK
SUB = 80
NSUB = CHUNK // SUB

_mesh = plsc.VectorSubcoreMesh(core_axis_name="c", subcore_axis_name="s")
_sc_params = pltpu.CompilerParams(needs_layout_passes=False)


def _stage_idx2d(idx1d, idx2d):
    for j in range(NSUB):
        for t in range(SUB // 16):
            idx2d[j, pl.ds(t * 16, 16)] = idx1d[pl.ds(j * SUB + t * 16, 16)]


@functools.partial(
    pl.kernel,
    out_type=(
        jax.ShapeDtypeStruct((NUM_CORES, S, C), jnp.float32),
        jax.ShapeDtypeStruct((NW * S,), jnp.float32),
    ),
    mesh=_mesh,
    scratch_types=[
        pltpu.VMEM((CHUNK, C), jnp.float32),
        pltpu.VMEM((CHUNK,), jnp.int32),
        pltpu.VMEM((NSUB, SUB), jnp.int32),
        pltpu.VMEM((S,), jnp.float32),
        pltpu.VMEM_SHARED((S, C), jnp.float32),
    ],
    compiler_params=_sc_params,
)
def _phase_a(x_hbm, b_hbm, zeros_hbm, psums_hbm, pcnt_hbm,
             xbuf, idx1d, idx2d, cnt, sums_sh):
    cid = lax.axis_index("c")
    sid = lax.axis_index("s")
    wid = cid * NUM_SUBCORES + sid

    rows_per_tile = S // NUM_SUBCORES
    pltpu.sync_copy(zeros_hbm.at[pl.ds(sid * rows_per_tile, rows_per_tile)],
                    sums_sh.at[pl.ds(sid * rows_per_tile, rows_per_tile)])
    zeros16 = jnp.zeros((16,), jnp.float32)

    def zero_body(i, carry):
        cnt[pl.ds(i * 16, 16)] = zeros16
        return carry

    lax.fori_loop(0, S // 16, zero_body, 0)
    plsc.subcore_barrier()

    ones16 = jnp.ones((16,), jnp.float32)

    def chunk_body(k, carry):
        row0 = wid * ROWS_PER_W + k * CHUNK
        pltpu.sync_copy(x_hbm.at[pl.ds(row0, CHUNK)], xbuf)
        pltpu.sync_copy(b_hbm.at[pl.ds(row0, CHUNK)], idx1d)
        _stage_idx2d(idx1d, idx2d)
        for j in range(NSUB):
            pltpu.sync_copy(xbuf.at[pl.ds(j * SUB, SUB)],
                            sums_sh.at[idx2d.at[j]], add=True)
            for t in range(SUB // 16):
                idx16 = idx2d[j, pl.ds(t * 16, 16)]
                plsc.addupdate_scatter(cnt, [idx16], ones16)
        return carry

    lax.fori_loop(0, NCHUNK, chunk_body, 0)

    plsc.subcore_barrier()
    pltpu.sync_copy(sums_sh.at[pl.ds(sid * rows_per_tile, rows_per_tile)],
                    psums_hbm.at[cid, pl.ds(sid * rows_per_tile, rows_per_tile)])
    pltpu.sync_copy(cnt, pcnt_hbm.at[pl.ds(wid * S, S)])


@functools.partial(
    pl.kernel,
    out_type=jax.ShapeDtypeStruct((NUM_CORES, S, C), jnp.float32),
    mesh=_mesh,
    scratch_types=[
        pltpu.VMEM((CHUNK, C), jnp.float32),
        pltpu.VMEM((CHUNK, C), jnp.float32),
        pltpu.VMEM((CHUNK,), jnp.int32),
        pltpu.VMEM((NSUB, SUB), jnp.int32),
        pltpu.VMEM_SHARED((S, C), jnp.float32),
        pltpu.SemaphoreType.DMA,
    ],
    compiler_params=_sc_params,
)
def _phase_b(x_hbm, b_hbm, c_hbm, zeros_hbm, hpart_hbm,
             xbuf, crows, idx1d, idx2d, h_sh, sem):
    cid = lax.axis_index("c")
    sid = lax.axis_index("s")
    wid = cid * NUM_SUBCORES + sid

    rows_per_tile = S // NUM_SUBCORES
    pltpu.sync_copy(zeros_hbm.at[pl.ds(sid * rows_per_tile, rows_per_tile)],
                    h_sh.at[pl.ds(sid * rows_per_tile, rows_per_tile)])
    plsc.subcore_barrier()

    def chunk_body(k, carry):
        row0 = wid * ROWS_PER_W + k * CHUNK
        pltpu.sync_copy(x_hbm.at[pl.ds(row0, CHUNK)], xbuf)
        pltpu.sync_copy(b_hbm.at[pl.ds(row0, CHUNK)], idx1d)
        _stage_idx2d(idx1d, idx2d)
        for j in range(NSUB):
            pltpu.async_copy(c_hbm.at[idx2d.at[j]],
                             crows.at[pl.ds(j * SUB, SUB)], sem).wait()

        @plsc.parallel_loop(0, CHUNK, step=1, unroll=8)
        def row_body(r):
            xs = [xbuf[r, pl.ds(16 * q, 16)] for q in range(C // 16)]
            cs = [crows[r, pl.ds(16 * q, 16)] for q in range(C // 16)]
            acc = xs[0] * cs[0]
            for q in range(1, C // 16):
                acc = acc + xs[q] * cs[q]
            z = jnp.sum(acc)
            zv = jnp.full((16,), z, jnp.float32)
            gate = 1.0 / (1.0 + jnp.exp(-zv))
            for q in range(C // 16):
                xbuf[r, pl.ds(16 * q, 16)] = xs[q] * gate

        for j in range(NSUB):
            pltpu.sync_copy(xbuf.at[pl.ds(j * SUB, SUB)],
                            h_sh.at[idx2d.at[j]], add=True)
        return carry

    lax.fori_loop(0, NCHUNK, chunk_body, 0)

    plsc.subcore_barrier()
    pltpu.sync_copy(h_sh.at[pl.ds(sid * rows_per_tile, rows_per_tile)],
                    hpart_hbm.at[cid, pl.ds(sid * rows_per_tile, rows_per_tile)])


def _mid_body(ps_ref, pc_ref, w_ref, c_ref):
    sums = ps_ref[0] + ps_ref[1]
    counts = jnp.sum(pc_ref[...].reshape(NW, S), axis=0)
    mean = sums / jnp.maximum(counts, 1.0)[:, None]
    c_ref[...] = jnp.tanh(
        jnp.dot(mean, w_ref[...], preferred_element_type=jnp.float32))


def _add_body(hp_ref, out_ref):
    out_ref[...] = hp_ref[0] + hp_ref[1]


def kernel(x, batch, weight_c):
    batch = batch.astype(jnp.int32)
    zeros = jnp.zeros((S, C), jnp.float32)

    psums, pcnt = _phase_a(x, batch, zeros)

    c = pl.pallas_call(
        _mid_body,
        out_shape=jax.ShapeDtypeStruct((S, C), jnp.float32),
    )(psums, pcnt, weight_c)

    hpart = _phase_b(x, batch, c, zeros)

    h = pl.pallas_call(
        _add_body,
        out_shape=jax.ShapeDtypeStruct((S, C), jnp.float32),
    )(hpart)
    return h
